# baseline (speedup 1.0000x reference)
import jax
import jax.numpy as jnp
from jax import lax
from jax.experimental import pallas as pl

K = 8
W = 0.1
N = 2048
B = 8
FOLD = 1


def _dot_t(a, b):
    return lax.dot_general(a, b, (((1,), (1,)), ((), ())),
                           preferred_element_type=jnp.float32)


def _rowmin_kth(E, k):
    Ef = E
    w = E.shape[1]
    for _ in range(FOLD.bit_length() - 1):
        w //= 2
        Ef = jnp.minimum(Ef[:, :w], Ef[:, w:])
    m = jnp.full((E.shape[0], 1), -jnp.inf, dtype=E.dtype)
    for _ in range(k):
        cand = jnp.where(Ef > m, Ef, jnp.inf)
        m = jnp.min(cand, axis=1, keepdims=True)
    return m


def _augment(x):
    xb = x.astype(jnp.bfloat16)
    x0 = x[:, 0:1]
    x1 = x[:, 1:2]
    x2 = x[:, 2:3]
    aa = (x0 * x0 + x1 * x1 + x2 * x2).astype(jnp.bfloat16)
    ones = jnp.ones_like(aa)
    lh = jnp.concatenate([xb, ones], axis=1)
    rh = jnp.concatenate([-2.0 * xb, aa], axis=1)
    E = _dot_t(lh, rh)
    thr = _rowmin_kth(E, K)
    Msel = (E <= thr).astype(jnp.bfloat16)
    F = jnp.concatenate(
        [x,
         x0 * x0, x0 * x1, x0 * x2,
         x1 * x0, x1 * x1, x1 * x2,
         x2 * x0, x2 * x1, x2 * x2], axis=1).astype(jnp.bfloat16)
    S = lax.dot_general(Msel, F, (((1,), (0,)), ((), ())),
                        preferred_element_type=jnp.float32)
    xs = (x0, x1, x2)
    cols = []
    for a in range(3):
        for b in range(3):
            c = (S[:, 3 + 3 * a + b:4 + 3 * a + b]
                 - S[:, a:a + 1] * xs[b]
                 - xs[a] * S[:, b:b + 1]
                 + float(K) * xs[a] * xs[b])
            cols.append((W / float(K)) * c)
    return jnp.concatenate([x] + cols, axis=1)


def _step(x1_ref, x2_ref, d1_ref, d2_ref):
    x1 = x1_ref[0]
    x2 = x2_ref[0]
    s1b = _augment(x1).astype(jnp.bfloat16)
    s2b = _augment(x2).astype(jnp.bfloat16)
    s1f = s1b.astype(jnp.float32)
    s2f = s2b.astype(jnp.float32)
    sa = jnp.sum(s1f * s1f, axis=1, keepdims=True)
    sb = jnp.sum(s2f * s2f, axis=1, keepdims=True)
    sb_hi = sb.astype(jnp.bfloat16)
    sb_lo = (sb - sb_hi.astype(jnp.float32)).astype(jnp.bfloat16)
    ones = jnp.ones((N, 1), dtype=jnp.bfloat16)
    lh = jnp.concatenate([s1b, ones, ones], axis=1)
    rh = jnp.concatenate([-2.0 * s2b, sb_hi, sb_lo], axis=1)
    E12 = _dot_t(lh, rh)
    d1_ref[0] = jnp.min(E12, axis=1, keepdims=True) + sa
    d2_ref[0] = jnp.min(E12 + sa, axis=0, keepdims=True)


@jax.jit
def kernel(xyz1, xyz2):
    d1, d2 = pl.pallas_call(
        _step,
        grid=(B,),
        in_specs=[
            pl.BlockSpec((1, N, 3), lambda b: (b, 0, 0)),
            pl.BlockSpec((1, N, 3), lambda b: (b, 0, 0)),
        ],
        out_specs=[
            pl.BlockSpec((1, N, 1), lambda b: (b, 0, 0)),
            pl.BlockSpec((1, 1, N), lambda b: (b, 0, 0)),
        ],
        out_shape=[
            jax.ShapeDtypeStruct((B, N, 1), jnp.float32),
            jax.ShapeDtypeStruct((B, 1, N), jnp.float32),
        ],
    )(xyz1, xyz2)
    return jnp.mean(d1) + jnp.mean(d2)


# speedup vs baseline: 1.0013x; 1.0013x over previous
import jax
import jax.numpy as jnp
from jax import lax
from jax.experimental import pallas as pl

K = 8
W = 0.1
N = 2048
B = 8
FOLD = 1


def _dot_t(a, b):
    return lax.dot_general(a, b, (((1,), (1,)), ((), ())),
                           preferred_element_type=jnp.float32)


def _rowmin_kth(E, k):
    Ef = E
    w = E.shape[1]
    for _ in range(FOLD.bit_length() - 1):
        w //= 2
        Ef = jnp.minimum(Ef[:, :w], Ef[:, w:])
    m = jnp.full((E.shape[0], 1), -jnp.inf, dtype=E.dtype)
    for _ in range(k):
        cand = jnp.where(Ef > m, Ef, jnp.inf)
        m = jnp.min(cand, axis=1, keepdims=True)
    return m


def _augment(x):
    xb = x.astype(jnp.bfloat16)
    x0 = x[:, 0:1]
    x1 = x[:, 1:2]
    x2 = x[:, 2:3]
    aa = (x0 * x0 + x1 * x1 + x2 * x2).astype(jnp.bfloat16)
    ones = jnp.ones_like(aa)
    lh = jnp.concatenate([xb, ones], axis=1)
    rh = jnp.concatenate([-2.0 * xb, aa], axis=1)
    E = _dot_t(lh, rh)
    thr = _rowmin_kth(E, K)
    Msel = (E <= thr).astype(jnp.bfloat16)
    F = jnp.concatenate(
        [x,
         x0 * x0, x0 * x1, x0 * x2,
         x1 * x0, x1 * x1, x1 * x2,
         x2 * x0, x2 * x1, x2 * x2], axis=1).astype(jnp.bfloat16)
    S = lax.dot_general(Msel, F, (((1,), (0,)), ((), ())),
                        preferred_element_type=jnp.float32)
    xs = (x0, x1, x2)
    cols = []
    for a in range(3):
        for b in range(3):
            c = (S[:, 3 + 3 * a + b:4 + 3 * a + b]
                 - S[:, a:a + 1] * xs[b]
                 - xs[a] * S[:, b:b + 1]
                 + float(K) * xs[a] * xs[b])
            cols.append((W / float(K)) * c)
    return jnp.concatenate([x] + cols, axis=1)


def _step(x1_ref, x2_ref, d1_ref, d2_ref):
    x1 = x1_ref[0]
    x2 = x2_ref[0]
    s1 = _augment(x1)
    s2 = _augment(x2)
    s1b = s1.astype(jnp.bfloat16)
    s2b = s2.astype(jnp.bfloat16)
    sa = jnp.sum(s1 * s1, axis=1, keepdims=True)
    sb = jnp.sum(s2 * s2, axis=1, keepdims=True)
    sb_hi = sb.astype(jnp.bfloat16)
    sb_lo = (sb - sb_hi.astype(jnp.float32)).astype(jnp.bfloat16)
    ones = jnp.ones((N, 1), dtype=jnp.bfloat16)
    lh = jnp.concatenate([s1b, ones, ones], axis=1)
    rh = jnp.concatenate([-2.0 * s2b, sb_hi, sb_lo], axis=1)
    E12 = _dot_t(lh, rh)
    d1_ref[0] = jnp.min(E12, axis=1, keepdims=True) + sa
    d2_ref[0] = jnp.min(E12 + sa, axis=0, keepdims=True)


@jax.jit
def kernel(xyz1, xyz2):
    d1, d2 = pl.pallas_call(
        _step,
        grid=(B,),
        in_specs=[
            pl.BlockSpec((1, N, 3), lambda b: (b, 0, 0)),
            pl.BlockSpec((1, N, 3), lambda b: (b, 0, 0)),
        ],
        out_specs=[
            pl.BlockSpec((1, N, 1), lambda b: (b, 0, 0)),
            pl.BlockSpec((1, 1, N), lambda b: (b, 0, 0)),
        ],
        out_shape=[
            jax.ShapeDtypeStruct((B, N, 1), jnp.float32),
            jax.ShapeDtypeStruct((B, 1, N), jnp.float32),
        ],
    )(xyz1, xyz2)
    return jnp.mean(d1) + jnp.mean(d2)


# speedup vs baseline: 1.3184x; 1.3167x over previous
import jax
import jax.numpy as jnp
from jax import lax
from jax.experimental import pallas as pl

K = 8
W = 0.1
N = 2048
B = 8
FOLD = 4


def _dot_t(a, b):
    return lax.dot_general(a, b, (((1,), (1,)), ((), ())),
                           preferred_element_type=jnp.float32)


def _rowmin_kth(E, k):
    Ef = E
    w = E.shape[1]
    for _ in range(FOLD.bit_length() - 1):
        w //= 2
        Ef = jnp.minimum(Ef[:, :w], Ef[:, w:])
    m = jnp.min(Ef, axis=1, keepdims=True)
    for _ in range(k - 1):
        cand = jnp.where(Ef > m, Ef, jnp.inf)
        m = jnp.min(cand, axis=1, keepdims=True)
    return m


def _augment(x):
    xb = x.astype(jnp.bfloat16)
    x0 = x[:, 0:1]
    x1 = x[:, 1:2]
    x2 = x[:, 2:3]
    aa = (x0 * x0 + x1 * x1 + x2 * x2).astype(jnp.bfloat16)
    ones = jnp.ones_like(aa)
    lh = jnp.concatenate([xb, ones], axis=1)
    rh = jnp.concatenate([-2.0 * xb, aa], axis=1)
    E = _dot_t(lh, rh)
    thr = _rowmin_kth(E, K)
    Msel = (E <= thr).astype(jnp.bfloat16)
    F = jnp.concatenate(
        [x,
         x0 * x0, x0 * x1, x0 * x2,
         x1 * x0, x1 * x1, x1 * x2,
         x2 * x0, x2 * x1, x2 * x2,
         jnp.ones_like(x0)], axis=1).astype(jnp.bfloat16)
    S = lax.dot_general(Msel, F, (((1,), (0,)), ((), ())),
                        preferred_element_type=jnp.float32)
    cnt = S[:, 12:13]
    xs = (x0, x1, x2)
    cols = []
    for a in range(3):
        for b in range(3):
            c = (S[:, 3 + 3 * a + b:4 + 3 * a + b]
                 - S[:, a:a + 1] * xs[b]
                 - xs[a] * S[:, b:b + 1]
                 + cnt * xs[a] * xs[b])
            cols.append((W / float(K)) * c)
    return jnp.concatenate([x] + cols, axis=1)


def _step(x1_ref, x2_ref, d1_ref, d2_ref):
    x1 = x1_ref[0]
    x2 = x2_ref[0]
    s1 = _augment(x1)
    s2 = _augment(x2)
    s1b = s1.astype(jnp.bfloat16)
    s2b = s2.astype(jnp.bfloat16)
    sa = jnp.sum(s1 * s1, axis=1, keepdims=True)
    sb = jnp.sum(s2 * s2, axis=1, keepdims=True)
    sa_hi = sa.astype(jnp.bfloat16)
    sa_lo = (sa - sa_hi.astype(jnp.float32)).astype(jnp.bfloat16)
    sb_hi = sb.astype(jnp.bfloat16)
    sb_lo = (sb - sb_hi.astype(jnp.float32)).astype(jnp.bfloat16)
    ones = jnp.ones((N, 1), dtype=jnp.bfloat16)
    lh = jnp.concatenate([s1b, ones, ones, sa_hi, sa_lo], axis=1)
    rh = jnp.concatenate([-2.0 * s2b, sb_hi, sb_lo, ones, ones], axis=1)
    D12 = _dot_t(lh, rh)
    d1_ref[0] = jnp.min(D12, axis=1, keepdims=True)
    d2_ref[0] = jnp.min(D12, axis=0, keepdims=True)


@jax.jit
def kernel(xyz1, xyz2):
    d1, d2 = pl.pallas_call(
        _step,
        grid=(B,),
        in_specs=[
            pl.BlockSpec((1, N, 3), lambda b: (b, 0, 0)),
            pl.BlockSpec((1, N, 3), lambda b: (b, 0, 0)),
        ],
        out_specs=[
            pl.BlockSpec((1, N, 1), lambda b: (b, 0, 0)),
            pl.BlockSpec((1, 1, N), lambda b: (b, 0, 0)),
        ],
        out_shape=[
            jax.ShapeDtypeStruct((B, N, 1), jnp.float32),
            jax.ShapeDtypeStruct((B, 1, N), jnp.float32),
        ],
    )(xyz1, xyz2)
    return jnp.mean(d1) + jnp.mean(d2)


# speedup vs baseline: 1.3735x; 1.0417x over previous
import jax
import jax.numpy as jnp
from jax import lax
from jax.experimental import pallas as pl

K = 8
W = 0.1
N = 2048
B = 8
FOLD = 4


def _dot_t(a, b, out_dtype=jnp.float32):
    return lax.dot_general(a, b, (((1,), (1,)), ((), ())),
                           preferred_element_type=out_dtype)


def _rowmin_kth(E, k):
    w = E.shape[1] // FOLD
    Ef = E[:, 0:w]
    for s in range(1, FOLD):
        Ef = jnp.minimum(Ef, E[:, s * w:(s + 1) * w])
    m = jnp.min(Ef, axis=1, keepdims=True)
    inf = jnp.array(jnp.inf, dtype=E.dtype)
    for _ in range(k - 1):
        cand = jnp.where(Ef > m, Ef, inf)
        m = jnp.min(cand, axis=1, keepdims=True)
    return m


def _augment(x):
    xb = x.astype(jnp.bfloat16)
    x0 = x[:, 0:1]
    x1 = x[:, 1:2]
    x2 = x[:, 2:3]
    aa = x0 * x0 + x1 * x1 + x2 * x2
    aa_hi = aa.astype(jnp.bfloat16)
    aa_lo = (aa - aa_hi.astype(jnp.float32)).astype(jnp.bfloat16)
    ones = jnp.ones_like(aa_hi)
    lh = jnp.concatenate([xb, ones, ones, aa_hi, aa_lo], axis=1)
    rh = jnp.concatenate([-2.0 * xb, aa_hi, aa_lo, ones, ones], axis=1)
    Dk = _dot_t(lh, rh).astype(jnp.bfloat16)
    thr = _rowmin_kth(Dk, K)
    Msel = (Dk <= thr).astype(jnp.bfloat16)
    F = jnp.concatenate(
        [x,
         x0 * x0, x0 * x1, x0 * x2,
         x1 * x0, x1 * x1, x1 * x2,
         x2 * x0, x2 * x1, x2 * x2,
         jnp.ones_like(x0)], axis=1).astype(jnp.bfloat16)
    S = lax.dot_general(Msel, F, (((1,), (0,)), ((), ())),
                        preferred_element_type=jnp.float32)
    cnt = S[:, 12:13]
    xs = (x0, x1, x2)
    cols = []
    for a in range(3):
        for b in range(3):
            c = (S[:, 3 + 3 * a + b:4 + 3 * a + b]
                 - S[:, a:a + 1] * xs[b]
                 - xs[a] * S[:, b:b + 1]
                 + cnt * xs[a] * xs[b])
            cols.append((W / float(K)) * c)
    return jnp.concatenate([x] + cols, axis=1)


def _step(x1_ref, x2_ref, d1_ref, d2_ref):
    x1 = x1_ref[0]
    x2 = x2_ref[0]
    s1 = _augment(x1)
    s2 = _augment(x2)
    s1b = s1.astype(jnp.bfloat16)
    s2b = s2.astype(jnp.bfloat16)
    sa = jnp.sum(s1 * s1, axis=1, keepdims=True)
    sb = jnp.sum(s2 * s2, axis=1, keepdims=True)
    sa_hi = sa.astype(jnp.bfloat16)
    sa_lo = (sa - sa_hi.astype(jnp.float32)).astype(jnp.bfloat16)
    sb_hi = sb.astype(jnp.bfloat16)
    sb_lo = (sb - sb_hi.astype(jnp.float32)).astype(jnp.bfloat16)
    ones = jnp.ones((N, 1), dtype=jnp.bfloat16)
    lh = jnp.concatenate([s1b, ones, ones, sa_hi, sa_lo], axis=1)
    rh = jnp.concatenate([-2.0 * s2b, sb_hi, sb_lo, ones, ones], axis=1)
    D12 = _dot_t(lh, rh)
    d1_ref[0] = jnp.min(D12, axis=1, keepdims=True)
    d2_ref[0] = jnp.min(D12, axis=0, keepdims=True)


@jax.jit
def kernel(xyz1, xyz2):
    d1, d2 = pl.pallas_call(
        _step,
        grid=(B,),
        in_specs=[
            pl.BlockSpec((1, N, 3), lambda b: (b, 0, 0)),
            pl.BlockSpec((1, N, 3), lambda b: (b, 0, 0)),
        ],
        out_specs=[
            pl.BlockSpec((1, N, 1), lambda b: (b, 0, 0)),
            pl.BlockSpec((1, 1, N), lambda b: (b, 0, 0)),
        ],
        out_shape=[
            jax.ShapeDtypeStruct((B, N, 1), jnp.float32),
            jax.ShapeDtypeStruct((B, 1, N), jnp.float32),
        ],
    )(xyz1, xyz2)
    return jnp.mean(d1) + jnp.mean(d2)


# speedup vs baseline: 2.3209x; 1.6898x over previous
import jax
import jax.numpy as jnp
from jax import lax
from jax.experimental import pallas as pl

K = 8
W = 0.1
N = 2048
B = 8
FOLD = 4


def _dot_t(a, b, out_dtype=jnp.float32):
    return lax.dot_general(a, b, (((1,), (1,)), ((), ())),
                           preferred_element_type=out_dtype)


def _rowmin_kth(E, k):
    w = E.shape[1] // FOLD
    Ef = E[:, 0:w]
    for s in range(1, FOLD):
        Ef = jnp.minimum(Ef, E[:, s * w:(s + 1) * w])
    m = jnp.min(Ef, axis=1, keepdims=True)
    inf = jnp.array(jnp.inf, dtype=E.dtype)
    for _ in range(k - 1):
        cand = jnp.where(Ef > m, Ef, inf)
        m = jnp.min(cand, axis=1, keepdims=True)
    return m


def _knn_dist(x):
    xb = x.astype(jnp.bfloat16)
    aa = jnp.sum(x * x, axis=1, keepdims=True)
    aa_hi = aa.astype(jnp.bfloat16)
    aa_lo = (aa - aa_hi.astype(jnp.float32)).astype(jnp.bfloat16)
    ones = jnp.ones_like(aa_hi)
    lh = jnp.concatenate([xb, ones, ones, aa_hi, aa_lo], axis=1)
    rh = jnp.concatenate([-2.0 * xb, aa_hi, aa_lo, ones, ones], axis=1)
    return _dot_t(lh, rh).astype(jnp.bfloat16)


def _neighbor_sums(Dk, x):
    thr = _rowmin_kth(Dk, K)
    Msel = (Dk <= thr).astype(jnp.bfloat16)
    x0 = x[:, 0:1]
    x1 = x[:, 1:2]
    x2 = x[:, 2:3]
    F = jnp.concatenate(
        [x,
         x0 * x0, x0 * x1, x0 * x2,
         x1 * x0, x1 * x1, x1 * x2,
         x2 * x0, x2 * x1, x2 * x2,
         jnp.ones_like(x0)], axis=1).astype(jnp.bfloat16)
    return lax.dot_general(Msel, F, (((1,), (0,)), ((), ())),
                           preferred_element_type=jnp.float32)


def _cov_features(S, x):
    cnt = S[:, 12:13]
    xs = (x[:, 0:1], x[:, 1:2], x[:, 2:3])
    cols = []
    for a in range(3):
        for b in range(3):
            c = (S[:, 3 + 3 * a + b:4 + 3 * a + b]
                 - S[:, a:a + 1] * xs[b]
                 - xs[a] * S[:, b:b + 1]
                 + cnt * xs[a] * xs[b])
            cols.append((W / float(K)) * c)
    return jnp.concatenate([x] + cols, axis=1)


def _augment(x):
    return _cov_features(_neighbor_sums(_knn_dist(x), x), x)


def _step(x1_ref, x2_ref, d1_ref, d2_ref):
    x1 = x1_ref[0]
    x2 = x2_ref[0]
    Dk1 = _knn_dist(x1)
    Dk2 = _knn_dist(x2)
    S1 = _neighbor_sums(Dk1, x1)
    S2 = _neighbor_sums(Dk2, x2)
    s1 = _cov_features(S1, x1)
    s2 = _cov_features(S2, x2)
    s1b = s1.astype(jnp.bfloat16)
    s2b = s2.astype(jnp.bfloat16)
    sa = jnp.sum(s1 * s1, axis=1, keepdims=True)
    sb = jnp.sum(s2 * s2, axis=1, keepdims=True)
    sa_hi = sa.astype(jnp.bfloat16)
    sa_lo = (sa - sa_hi.astype(jnp.float32)).astype(jnp.bfloat16)
    sb_hi = sb.astype(jnp.bfloat16)
    sb_lo = (sb - sb_hi.astype(jnp.float32)).astype(jnp.bfloat16)
    ones = jnp.ones((N, 1), dtype=jnp.bfloat16)
    lh = jnp.concatenate([s1b, ones, ones, sa_hi, sa_lo], axis=1)
    rh = jnp.concatenate([-2.0 * s2b, sb_hi, sb_lo, ones, ones], axis=1)
    D12 = _dot_t(lh, rh)
    d1_ref[0] = jnp.min(D12, axis=1, keepdims=True)
    d2_ref[0] = jnp.min(D12, axis=0, keepdims=True)


@jax.jit
def kernel(xyz1, xyz2):
    d1, d2 = pl.pallas_call(
        _step,
        grid=(B,),
        in_specs=[
            pl.BlockSpec((1, N, 3), lambda b: (b, 0, 0)),
            pl.BlockSpec((1, N, 3), lambda b: (b, 0, 0)),
        ],
        out_specs=[
            pl.BlockSpec((1, N, 1), lambda b: (b, 0, 0)),
            pl.BlockSpec((1, 1, N), lambda b: (b, 0, 0)),
        ],
        out_shape=[
            jax.ShapeDtypeStruct((B, N, 1), jnp.float32),
            jax.ShapeDtypeStruct((B, 1, N), jnp.float32),
        ],
    )(xyz1, xyz2)
    return jnp.mean(d1) + jnp.mean(d2)


# speedup vs baseline: 2.3491x; 1.0121x over previous
import jax
import jax.numpy as jnp
from jax import lax
from jax.experimental import pallas as pl

K = 8
W = 0.1
N = 2048
B = 8
FOLD = 8


def _dot_t(a, b, out_dtype=jnp.float32):
    return lax.dot_general(a, b, (((1,), (1,)), ((), ())),
                           preferred_element_type=out_dtype)


def _rowmin_kth(E, k):
    w = E.shape[1] // FOLD
    Ef = E[:, 0:w]
    for s in range(1, FOLD):
        Ef = jnp.minimum(Ef, E[:, s * w:(s + 1) * w])
    m = jnp.min(Ef, axis=1, keepdims=True)
    inf = jnp.array(jnp.inf, dtype=E.dtype)
    for _ in range(k - 1):
        cand = jnp.where(Ef > m, Ef, inf)
        m = jnp.min(cand, axis=1, keepdims=True)
    return m


def _knn_dist(x):
    xb = x.astype(jnp.bfloat16)
    aa = jnp.sum(x * x, axis=1, keepdims=True)
    aa_hi = aa.astype(jnp.bfloat16)
    aa_lo = (aa - aa_hi.astype(jnp.float32)).astype(jnp.bfloat16)
    ones = jnp.ones_like(aa_hi)
    lh = jnp.concatenate([xb, ones, ones, aa_hi, aa_lo], axis=1)
    rh = jnp.concatenate([-2.0 * xb, aa_hi, aa_lo, ones, ones], axis=1)
    return _dot_t(lh, rh).astype(jnp.bfloat16)


def _neighbor_sums(Dk, x):
    thr = _rowmin_kth(Dk, K)
    Msel = (Dk <= thr).astype(jnp.bfloat16)
    x0 = x[:, 0:1]
    x1 = x[:, 1:2]
    x2 = x[:, 2:3]
    F = jnp.concatenate(
        [x,
         x0 * x0, x0 * x1, x0 * x2,
         x1 * x0, x1 * x1, x1 * x2,
         x2 * x0, x2 * x1, x2 * x2,
         jnp.ones_like(x0)], axis=1).astype(jnp.bfloat16)
    return lax.dot_general(Msel, F, (((1,), (0,)), ((), ())),
                           preferred_element_type=jnp.float32)


def _cov_features(S, x):
    cnt = S[:, 12:13]
    xs = (x[:, 0:1], x[:, 1:2], x[:, 2:3])
    cols = []
    for a in range(3):
        for b in range(3):
            c = (S[:, 3 + 3 * a + b:4 + 3 * a + b]
                 - S[:, a:a + 1] * xs[b]
                 - xs[a] * S[:, b:b + 1]
                 + cnt * xs[a] * xs[b])
            cols.append((W / float(K)) * c)
    return jnp.concatenate([x] + cols, axis=1)


def _augment(x):
    return _cov_features(_neighbor_sums(_knn_dist(x), x), x)


def _step(x1_ref, x2_ref, d1_ref, d2_ref):
    x1 = x1_ref[0]
    x2 = x2_ref[0]
    Dk1 = _knn_dist(x1)
    Dk2 = _knn_dist(x2)
    S1 = _neighbor_sums(Dk1, x1)
    S2 = _neighbor_sums(Dk2, x2)
    s1 = _cov_features(S1, x1)
    s2 = _cov_features(S2, x2)
    s1b = s1.astype(jnp.bfloat16)
    s2b = s2.astype(jnp.bfloat16)
    sa = jnp.sum(s1 * s1, axis=1, keepdims=True)
    sb = jnp.sum(s2 * s2, axis=1, keepdims=True)
    sa_hi = sa.astype(jnp.bfloat16)
    sa_lo = (sa - sa_hi.astype(jnp.float32)).astype(jnp.bfloat16)
    sb_hi = sb.astype(jnp.bfloat16)
    sb_lo = (sb - sb_hi.astype(jnp.float32)).astype(jnp.bfloat16)
    ones = jnp.ones((N, 1), dtype=jnp.bfloat16)
    lh = jnp.concatenate([s1b, ones, ones, sa_hi, sa_lo], axis=1)
    rh = jnp.concatenate([-2.0 * s2b, sb_hi, sb_lo, ones, ones], axis=1)
    D12 = _dot_t(lh, rh)
    d1_ref[0] = jnp.min(D12, axis=1, keepdims=True)
    d2_ref[0] = jnp.min(D12, axis=0, keepdims=True)


@jax.jit
def kernel(xyz1, xyz2):
    d1, d2 = pl.pallas_call(
        _step,
        grid=(B,),
        in_specs=[
            pl.BlockSpec((1, N, 3), lambda b: (b, 0, 0)),
            pl.BlockSpec((1, N, 3), lambda b: (b, 0, 0)),
        ],
        out_specs=[
            pl.BlockSpec((1, N, 1), lambda b: (b, 0, 0)),
            pl.BlockSpec((1, 1, N), lambda b: (b, 0, 0)),
        ],
        out_shape=[
            jax.ShapeDtypeStruct((B, N, 1), jnp.float32),
            jax.ShapeDtypeStruct((B, 1, N), jnp.float32),
        ],
    )(xyz1, xyz2)
    return jnp.mean(d1) + jnp.mean(d2)
